# Initial kernel scaffold; baseline (speedup 1.0000x reference)
import jax
import jax.numpy as jnp
from jax.experimental import pallas as pl


def kernel(x, weight):
    raise NotImplementedError("write your pallas kernel here")


# speedup vs baseline: 4.8582x; 4.8582x over previous
import functools

import jax
import jax.numpy as jnp
from jax import lax
from jax.experimental import pallas as pl
from jax.experimental.pallas import tpu as pltpu
from jax.experimental.pallas import tpu_sc as plsc

VOCAB = 1000
EMBED = 128
BATCH = 4096
SEQ = 200
N = BATCH * SEQ

NC = 2
NS = 16
NW = NC * NS
B_PER_W = N // NW
CHUNK = 128
STEPS = B_PER_W // CHUNK


@functools.partial(
    pl.kernel,
    out_type=jax.ShapeDtypeStruct((N, EMBED), jnp.float32),
    mesh=plsc.VectorSubcoreMesh(core_axis_name="c", subcore_axis_name="s"),
    scratch_types=[
        pltpu.VMEM((CHUNK,), jnp.int32),
        pltpu.VMEM((CHUNK, EMBED), jnp.float32),
        pltpu.SemaphoreType.DMA,
    ],
)
def _embed_lookup(x_hbm, w_hbm, out_hbm, idx_v, rows_v, sem):
    wid = lax.axis_index("s") * NC + lax.axis_index("c")
    base = wid * B_PER_W

    def step(g, carry):
        off = base + g * CHUNK
        pltpu.sync_copy(x_hbm.at[pl.ds(off, CHUNK)], idx_v)
        pltpu.async_copy(w_hbm.at[idx_v], rows_v, sem).wait()
        pltpu.sync_copy(rows_v, out_hbm.at[pl.ds(off, CHUNK)])
        return carry

    lax.fori_loop(0, STEPS, step, 0)


def kernel(x, weight):
    xf = x.reshape(N).astype(jnp.int32)
    out = _embed_lookup(xf, weight)
    return out.reshape(BATCH, SEQ, EMBED)


# speedup vs baseline: 6.6765x; 1.3743x over previous
import functools

import jax
import jax.numpy as jnp
from jax import lax
from jax.experimental import pallas as pl
from jax.experimental.pallas import tpu as pltpu
from jax.experimental.pallas import tpu_sc as plsc

VOCAB = 1000
EMBED = 128
BATCH = 4096
SEQ = 200
N = BATCH * SEQ

NC = 2
NS = 16
NW = NC * NS
B_PER_W = N // NW
CHUNK = 128
K = 4
GROUP = K * CHUNK
NGROUPS = B_PER_W // GROUP


@functools.partial(
    pl.kernel,
    out_type=jax.ShapeDtypeStruct((N, EMBED), jnp.float32),
    mesh=plsc.VectorSubcoreMesh(core_axis_name="c", subcore_axis_name="s"),
    scratch_types=(
        [pltpu.VMEM((GROUP,), jnp.int32)]
        + [pltpu.VMEM((CHUNK, EMBED), jnp.float32) for _ in range(K)]
        + [pltpu.SemaphoreType.DMA for _ in range(2 * K)]
    ),
)
def _embed_lookup(x_hbm, w_hbm, out_hbm, idx_v, *bufs_and_sems):
    rows = bufs_and_sems[:K]
    gsem = bufs_and_sems[K:2 * K]
    ssem = bufs_and_sems[2 * K:]
    wid = lax.axis_index("s") * NC + lax.axis_index("c")
    base = wid * B_PER_W

    def step(g, carry):
        off = base + g * GROUP

        @pl.when(g > 0)
        def _():
            for j in range(K):
                pltpu.make_async_copy(
                    rows[j], out_hbm.at[pl.ds(off, CHUNK)], ssem[j]
                ).wait()

        pltpu.sync_copy(x_hbm.at[pl.ds(off, GROUP)], idx_v)
        for j in range(K):
            pltpu.async_copy(
                w_hbm.at[idx_v.at[pl.ds(j * CHUNK, CHUNK)]], rows[j], gsem[j]
            )
        for j in range(K):
            pltpu.make_async_copy(
                w_hbm.at[idx_v.at[pl.ds(j * CHUNK, CHUNK)]], rows[j], gsem[j]
            ).wait()
            pltpu.async_copy(
                rows[j], out_hbm.at[pl.ds(off + j * CHUNK, CHUNK)], ssem[j]
            )
        return carry

    lax.fori_loop(0, NGROUPS, step, 0)
    for j in range(K):
        pltpu.make_async_copy(
            rows[j], out_hbm.at[pl.ds(base, CHUNK)], ssem[j]
        ).wait()


def kernel(x, weight):
    xf = x.reshape(N).astype(jnp.int32)
    out = _embed_lookup(xf, weight)
    return out.reshape(BATCH, SEQ, EMBED)


# speedup vs baseline: 6.7453x; 1.0103x over previous
import functools

import jax
import jax.numpy as jnp
from jax import lax
from jax.experimental import pallas as pl
from jax.experimental.pallas import tpu as pltpu
from jax.experimental.pallas import tpu_sc as plsc

VOCAB = 1000
EMBED = 128
BATCH = 4096
SEQ = 200
N = BATCH * SEQ

NC = 2
NS = 16
NW = NC * NS
B_PER_W = N // NW
CHUNK = 128
K = 2
SET = K * CHUNK
BODY = 2 * SET
NB = B_PER_W // BODY


@functools.partial(
    pl.kernel,
    out_type=jax.ShapeDtypeStruct((N, EMBED), jnp.float32),
    mesh=plsc.VectorSubcoreMesh(core_axis_name="c", subcore_axis_name="s"),
    scratch_types=(
        [pltpu.VMEM((SET,), jnp.int32) for _ in range(2)]
        + [pltpu.VMEM((CHUNK, EMBED), jnp.float32) for _ in range(2 * K)]
        + [pltpu.SemaphoreType.DMA for _ in range(2 + 4 * K)]
    ),
)
def _embed_lookup(x_hbm, w_hbm, out_hbm, idx_a, idx_b, *bufs_and_sems):
    rows_a = bufs_and_sems[:K]
    rows_b = bufs_and_sems[K:2 * K]
    sems = bufs_and_sems[2 * K:]
    isem_a, isem_b = sems[0], sems[1]
    gsem_a = sems[2:2 + K]
    gsem_b = sems[2 + K:2 + 2 * K]
    ssem_a = sems[2 + 2 * K:2 + 3 * K]
    ssem_b = sems[2 + 3 * K:2 + 4 * K]

    wid = lax.axis_index("s") * NC + lax.axis_index("c")
    base = wid * B_PER_W

    def idx_slice(buf, j):
        return buf.at[pl.ds(j * CHUNK, CHUNK)]

    pltpu.async_copy(x_hbm.at[pl.ds(base, SET)], idx_a, isem_a)

    def step(i, carry):
        off_a = base + i * BODY
        off_b = off_a + SET

        pltpu.async_copy(x_hbm.at[pl.ds(off_b, SET)], idx_b, isem_b)
        pltpu.make_async_copy(x_hbm.at[pl.ds(off_a, SET)], idx_a, isem_a).wait()

        for j in range(K):
            @pl.when(i > 0)
            def _():
                pltpu.make_async_copy(
                    rows_a[j], out_hbm.at[pl.ds(off_a, CHUNK)], ssem_a[j]
                ).wait()
            pltpu.async_copy(w_hbm.at[idx_slice(idx_a, j)], rows_a[j], gsem_a[j])

        pltpu.make_async_copy(x_hbm.at[pl.ds(off_b, SET)], idx_b, isem_b).wait()
        for j in range(K):
            @pl.when(i > 0)
            def _():
                pltpu.make_async_copy(
                    rows_b[j], out_hbm.at[pl.ds(off_b, CHUNK)], ssem_b[j]
                ).wait()
            pltpu.async_copy(w_hbm.at[idx_slice(idx_b, j)], rows_b[j], gsem_b[j])

        for j in range(K):
            pltpu.make_async_copy(
                w_hbm.at[idx_slice(idx_a, j)], rows_a[j], gsem_a[j]
            ).wait()
            pltpu.async_copy(
                rows_a[j], out_hbm.at[pl.ds(off_a + j * CHUNK, CHUNK)], ssem_a[j]
            )

        @pl.when(i < NB - 1)
        def _():
            pltpu.async_copy(
                x_hbm.at[pl.ds(off_a + BODY, SET)], idx_a, isem_a
            )

        for j in range(K):
            pltpu.make_async_copy(
                w_hbm.at[idx_slice(idx_b, j)], rows_b[j], gsem_b[j]
            ).wait()
            pltpu.async_copy(
                rows_b[j], out_hbm.at[pl.ds(off_b + j * CHUNK, CHUNK)], ssem_b[j]
            )
        return carry

    lax.fori_loop(0, NB, step, 0)

    for j in range(K):
        pltpu.make_async_copy(
            rows_a[j], out_hbm.at[pl.ds(base, CHUNK)], ssem_a[j]
        ).wait()
        pltpu.make_async_copy(
            rows_b[j], out_hbm.at[pl.ds(base, CHUNK)], ssem_b[j]
        ).wait()


def kernel(x, weight):
    xf = x.reshape(N).astype(jnp.int32)
    out = _embed_lookup(xf, weight)
    return out.reshape(BATCH, SEQ, EMBED)
